# Initial kernel scaffold; baseline (speedup 1.0000x reference)
import jax
import jax.numpy as jnp
from jax.experimental import pallas as pl


def kernel(node_features, edge_index, W1, b1, W2, b2, W_fc, b_fc):
    raise NotImplementedError("write your pallas kernel here")


# speedup vs baseline: 8.3273x; 8.3273x over previous
import functools

import jax
import jax.numpy as jnp
from jax import lax
from jax.experimental import pallas as pl
from jax.experimental.pallas import tpu as pltpu
from jax.experimental.pallas import tpu_sc as plsc

N = 10000
E = 320000
D = 128
TILES = 16
CORES = 2
CHUNK = 128
NCHUNK = 80
EPT = NCHUNK * CHUNK
E_PAD = EPT * TILES * CORES
N_ACC = 10240
PAD_DST = 10008
ROWS_T = 624
ROWS_LAST = N - 15 * ROWS_T
ACC_T = N_ACC // TILES

_MESH = plsc.VectorSubcoreMesh(core_axis_name="c", subcore_axis_name="s")


@functools.partial(
    pl.kernel,
    out_type=jax.ShapeDtypeStruct((CORES, N_ACC), jnp.float32),
    mesh=_MESH,
    scratch_types=[
        pltpu.VMEM((NCHUNK, CHUNK), jnp.int32),
        pltpu.VMEM((CHUNK,), jnp.float32),
        pltpu.VMEM((ACC_T,), jnp.float32),
        pltpu.VMEM_SHARED((N_ACC,), jnp.float32),
    ],
)
def _sc_degree(dst_hbm, deg_hbm, dst_v, ones_v, zer_v, deg_sh):
    c = lax.axis_index("c")
    s = lax.axis_index("s")
    for i in range(CHUNK // 16):
        ones_v[pl.ds(i * 16, 16)] = jnp.ones((16,), jnp.float32)
    def _z(i, _):
        zer_v[pl.ds(i * 16, 16)] = jnp.zeros((16,), jnp.float32)
        return 0
    lax.fori_loop(0, ACC_T // 16, _z, 0)
    pltpu.sync_copy(zer_v, deg_sh.at[pl.ds(s * ACC_T, ACC_T)])
    pltpu.sync_copy(
        dst_hbm.at[pl.ds((c * TILES + s) * NCHUNK, NCHUNK), :], dst_v)
    plsc.subcore_barrier()

    def _body(j, _):
        pltpu.sync_copy(ones_v, deg_sh.at[dst_v.at[j]], add=True)
        return 0
    lax.fori_loop(0, NCHUNK, _body, 0)
    plsc.subcore_barrier()
    pltpu.sync_copy(deg_sh.at[pl.ds(s * ACC_T, ACC_T)],
                    deg_hbm.at[c, pl.ds(s * ACC_T, ACC_T)])


@functools.partial(
    pl.kernel,
    out_type=jax.ShapeDtypeStruct((CORES, N, D), jnp.float32),
    mesh=_MESH,
    scratch_types=[
        pltpu.VMEM((EPT,), jnp.int32),
        pltpu.VMEM((NCHUNK, CHUNK), jnp.int32),
        pltpu.VMEM((CHUNK, D), jnp.float32),
        pltpu.VMEM((16, D), jnp.float32),
        pltpu.VMEM_SHARED((N_ACC, D), jnp.float32),
        pltpu.SemaphoreType.DMA,
    ],
)
def _sc_message(hp_hbm, src_hbm, dst_hbm, out_hbm,
                src_v, dst_v, rows_v, zer_v, acc_sh, sem):
    c = lax.axis_index("c")
    s = lax.axis_index("s")
    def _z(i, _):
        zer_v[i // 8, pl.ds((i % 8) * 16, 16)] = jnp.zeros((16,), jnp.float32)
        return 0
    lax.fori_loop(0, 16 * D // 16, _z, 0)

    def _zacc(j, _):
        pltpu.sync_copy(zer_v, acc_sh.at[pl.ds(s * ACC_T + j * 16, 16), :])
        return 0
    lax.fori_loop(0, ACC_T // 16, _zacc, 0)
    pltpu.sync_copy(src_hbm.at[pl.ds((c * TILES + s) * EPT, EPT)], src_v)
    pltpu.sync_copy(
        dst_hbm.at[pl.ds((c * TILES + s) * NCHUNK, NCHUNK), :], dst_v)
    plsc.subcore_barrier()

    def _body(j, _):
        idx = src_v.at[pl.ds(j * CHUNK, CHUNK)]
        pltpu.async_copy(hp_hbm.at[idx], rows_v, sem).wait()
        pltpu.sync_copy(rows_v, acc_sh.at[dst_v.at[j]], add=True)
        return 0
    lax.fori_loop(0, NCHUNK, _body, 0)
    plsc.subcore_barrier()

    @pl.when(s < 15)
    def _drain():
        pltpu.sync_copy(acc_sh.at[pl.ds(s * ROWS_T, ROWS_T), :],
                        out_hbm.at[c, pl.ds(s * ROWS_T, ROWS_T), :])

    @pl.when(s == 15)
    def _drain_last():
        pltpu.sync_copy(acc_sh.at[pl.ds(15 * ROWS_T, ROWS_LAST), :],
                        out_hbm.at[c, pl.ds(15 * ROWS_T, ROWS_LAST), :])


_BM = 1000


def _tc_first(x, W1, deg):
    def body(x_ref, w_ref, d_ref, o_ref):
        dinv = lax.rsqrt(d_ref[0] + d_ref[1] + 1.0)
        o_ref[...] = jnp.dot(x_ref[...], w_ref[...],
                             preferred_element_type=jnp.float32) * dinv
    return pl.pallas_call(
        body,
        grid=(N // _BM,),
        in_specs=[
            pl.BlockSpec((_BM, D), lambda i: (i, 0)),
            pl.BlockSpec((D, D), lambda i: (0, 0)),
            pl.BlockSpec((CORES, _BM, 1), lambda i: (0, i, 0)),
        ],
        out_specs=pl.BlockSpec((_BM, D), lambda i: (i, 0)),
        out_shape=jax.ShapeDtypeStruct((N, D), jnp.float32),
    )(x, W1, deg)


def _tc_mid(acc, hp, deg, b, W2):
    def body(a_ref, h_ref, d_ref, b_ref, w_ref, o_ref):
        dinv = lax.rsqrt(d_ref[0] + d_ref[1] + 1.0)
        tot = a_ref[0] + a_ref[1] + h_ref[...]
        x2 = jnp.maximum(tot * dinv + b_ref[...], 0.0)
        o_ref[...] = jnp.dot(x2, w_ref[...],
                             preferred_element_type=jnp.float32) * dinv
    return pl.pallas_call(
        body,
        grid=(N // _BM,),
        in_specs=[
            pl.BlockSpec((CORES, _BM, D), lambda i: (0, i, 0)),
            pl.BlockSpec((_BM, D), lambda i: (i, 0)),
            pl.BlockSpec((CORES, _BM, 1), lambda i: (0, i, 0)),
            pl.BlockSpec((1, D), lambda i: (0, 0)),
            pl.BlockSpec((D, D), lambda i: (0, 0)),
        ],
        out_specs=pl.BlockSpec((_BM, D), lambda i: (i, 0)),
        out_shape=jax.ShapeDtypeStruct((N, D), jnp.float32),
    )(acc, hp, deg, b, W2)


def _tc_last(acc, hp, deg, b, Wfm, Wfx, bfc):
    def body(a_ref, h_ref, d_ref, b_ref, wm_ref, wx_ref, bf_ref, o_ref,
             sum_ref, max_ref):
        i = pl.program_id(0)
        dinv = lax.rsqrt(d_ref[0] + d_ref[1] + 1.0)
        tot = a_ref[0] + a_ref[1] + h_ref[...]
        x3 = jnp.maximum(tot * dinv + b_ref[...], 0.0)
        bsum = jnp.sum(x3, axis=0, keepdims=True)
        bmax = jnp.max(x3, axis=0, keepdims=True)

        @pl.when(i == 0)
        def _init():
            sum_ref[...] = bsum
            max_ref[...] = bmax

        @pl.when(i > 0)
        def _accum():
            sum_ref[...] += bsum
            max_ref[...] = jnp.maximum(max_ref[...], bmax)

        @pl.when(i == N // _BM - 1)
        def _final():
            mean = sum_ref[...] * (1.0 / N)
            o_ref[...] = (jnp.dot(mean, wm_ref[...],
                                  preferred_element_type=jnp.float32)
                          + jnp.dot(max_ref[...], wx_ref[...],
                                    preferred_element_type=jnp.float32)
                          + bf_ref[...])
    return pl.pallas_call(
        body,
        grid=(N // _BM,),
        in_specs=[
            pl.BlockSpec((CORES, _BM, D), lambda i: (0, i, 0)),
            pl.BlockSpec((_BM, D), lambda i: (i, 0)),
            pl.BlockSpec((CORES, _BM, 1), lambda i: (0, i, 0)),
            pl.BlockSpec((1, D), lambda i: (0, 0)),
            pl.BlockSpec((D, D), lambda i: (0, 0)),
            pl.BlockSpec((D, D), lambda i: (0, 0)),
            pl.BlockSpec((1, D), lambda i: (0, 0)),
        ],
        out_specs=pl.BlockSpec((1, D), lambda i: (0, 0)),
        out_shape=jax.ShapeDtypeStruct((1, D), jnp.float32),
        scratch_shapes=[
            pltpu.VMEM((1, D), jnp.float32),
            pltpu.VMEM((1, D), jnp.float32),
        ],
    )(acc, hp, deg, b, Wfm, Wfx, bfc)


def kernel(node_features, edge_index, W1, b1, W2, b2, W_fc, b_fc):
    ei = edge_index.astype(jnp.int32)
    npad = E_PAD - E
    src = jnp.concatenate([ei[0], jnp.zeros((npad,), jnp.int32)])
    dst = jnp.concatenate([ei[1], jnp.full((npad,), PAD_DST, jnp.int32)])
    dst2d = dst.reshape(E_PAD // CHUNK, CHUNK)

    deg = _sc_degree(dst2d)[:, :N].reshape(CORES, N, 1)
    hp1 = _tc_first(node_features, W1, deg)
    acc1 = _sc_message(hp1, src, dst2d)
    hp2 = _tc_mid(acc1, hp1, deg, b1.reshape(1, D), W2)
    acc2 = _sc_message(hp2, src, dst2d)
    return _tc_last(acc2, hp2, deg, b2.reshape(1, D),
                    W_fc[:D], W_fc[D:], b_fc.reshape(1, D))


# speedup vs baseline: 8.9185x; 1.0710x over previous
import functools

import jax
import jax.numpy as jnp
from jax import lax
from jax.experimental import pallas as pl
from jax.experimental.pallas import tpu as pltpu
from jax.experimental.pallas import tpu_sc as plsc

N = 10000
E = 320000
D = 128
TILES = 16
CORES = 2
CHUNK = 128
GCHUNK = 256
NCHUNK = 80
NGCHUNK = 40
EPT = NCHUNK * CHUNK
E_PAD = EPT * TILES * CORES
N_ACC = 10240
PAD_DST = 10008
ROWS_T = 624
ROWS_LAST = N - 15 * ROWS_T
ACC_T = N_ACC // TILES

_MESH = plsc.VectorSubcoreMesh(core_axis_name="c", subcore_axis_name="s")


@functools.partial(
    pl.kernel,
    out_type=jax.ShapeDtypeStruct((CORES, N_ACC), jnp.float32),
    mesh=_MESH,
    scratch_types=[
        pltpu.VMEM((NCHUNK, CHUNK), jnp.int32),
        pltpu.VMEM((CHUNK,), jnp.float32),
        pltpu.VMEM((ACC_T,), jnp.float32),
        pltpu.VMEM_SHARED((N_ACC,), jnp.float32),
    ],
)
def _sc_degree(dst_hbm, deg_hbm, dst_v, ones_v, zer_v, deg_sh):
    c = lax.axis_index("c")
    s = lax.axis_index("s")
    for i in range(CHUNK // 16):
        ones_v[pl.ds(i * 16, 16)] = jnp.ones((16,), jnp.float32)
    def _z(i, _):
        zer_v[pl.ds(i * 16, 16)] = jnp.zeros((16,), jnp.float32)
        return 0
    lax.fori_loop(0, ACC_T // 16, _z, 0)
    pltpu.sync_copy(zer_v, deg_sh.at[pl.ds(s * ACC_T, ACC_T)])
    pltpu.sync_copy(
        dst_hbm.at[pl.ds((c * TILES + s) * NCHUNK, NCHUNK), :], dst_v)
    plsc.subcore_barrier()

    def _body(j, _):
        pltpu.sync_copy(ones_v, deg_sh.at[dst_v.at[j]], add=True)
        return 0
    lax.fori_loop(0, NCHUNK, _body, 0)
    plsc.subcore_barrier()
    pltpu.sync_copy(deg_sh.at[pl.ds(s * ACC_T, ACC_T)],
                    deg_hbm.at[c, pl.ds(s * ACC_T, ACC_T)])


@functools.partial(
    pl.kernel,
    out_type=jax.ShapeDtypeStruct((CORES, N, D), jnp.float32),
    mesh=_MESH,
    scratch_types=[
        pltpu.VMEM((CHUNK,), jnp.int32),
        pltpu.VMEM((CHUNK,), jnp.int32),
        pltpu.VMEM((NCHUNK, CHUNK), jnp.int32),
        pltpu.VMEM((CHUNK, D), jnp.float32),
        pltpu.VMEM((CHUNK, D), jnp.float32),
        pltpu.VMEM((8, D), jnp.float32),
        pltpu.VMEM_SHARED((N_ACC, D), jnp.float32),
        pltpu.SemaphoreType.DMA,
        pltpu.SemaphoreType.DMA,
    ],
)
def _sc_message(hp_hbm, src_hbm, dst_hbm, out_hbm,
                sidx0_v, sidx1_v, dst_v, rows0_v, rows1_v, zer_v, acc_sh,
                sem_g, sem_i):
    c = lax.axis_index("c")
    s = lax.axis_index("s")
    base = (c * TILES + s) * EPT
    def _z(i, _):
        zer_v[i // 8, pl.ds((i % 8) * 16, 16)] = jnp.zeros((16,), jnp.float32)
        return 0
    lax.fori_loop(0, 8 * D // 16, _z, 0)

    def _zacc(j, _):
        pltpu.sync_copy(zer_v, acc_sh.at[pl.ds(s * ACC_T + j * 8, 8), :])
        return 0
    lax.fori_loop(0, ACC_T // 8, _zacc, 0)
    pltpu.sync_copy(
        dst_hbm.at[pl.ds((c * TILES + s) * NCHUNK, NCHUNK), :], dst_v)
    plsc.subcore_barrier()

    def _fetch_idx(g, ibuf):
        pltpu.async_copy(src_hbm.at[pl.ds(base + g * CHUNK, CHUNK)],
                         ibuf, sem_i)

    def _iwait(ibuf):
        pltpu.make_async_copy(src_hbm.at[pl.ds(0, CHUNK)], ibuf, sem_i).wait()

    def _gather(ibuf, buf):
        pltpu.async_copy(hp_hbm.at[ibuf], buf, sem_g)

    def _gwait(ibuf, buf):
        pltpu.make_async_copy(hp_hbm.at[ibuf], buf, sem_g).wait()

    _fetch_idx(0, sidx0_v)
    _iwait(sidx0_v)
    _gather(sidx0_v, rows0_v)
    _fetch_idx(1, sidx1_v)

    def _body(g, _):
        def _step(cur_i, cur_r, nxt_i, nxt_r):
            _gwait(cur_i, cur_r)

            @pl.when(g + 1 < NCHUNK)
            def _launch_next():
                _iwait(nxt_i)
                _gather(nxt_i, nxt_r)

            @pl.when(g + 2 < NCHUNK)
            def _prefetch_idx():
                _fetch_idx(g + 2, cur_i)
            pltpu.sync_copy(cur_r, acc_sh.at[dst_v.at[g]], add=True)

        @pl.when(g % 2 == 0)
        def _even():
            _step(sidx0_v, rows0_v, sidx1_v, rows1_v)

        @pl.when(g % 2 == 1)
        def _odd():
            _step(sidx1_v, rows1_v, sidx0_v, rows0_v)
        return 0
    lax.fori_loop(0, NCHUNK, _body, 0)
    plsc.subcore_barrier()

    @pl.when(s < 15)
    def _drain():
        pltpu.sync_copy(acc_sh.at[pl.ds(s * ROWS_T, ROWS_T), :],
                        out_hbm.at[c, pl.ds(s * ROWS_T, ROWS_T), :])

    @pl.when(s == 15)
    def _drain_last():
        pltpu.sync_copy(acc_sh.at[pl.ds(15 * ROWS_T, ROWS_LAST), :],
                        out_hbm.at[c, pl.ds(15 * ROWS_T, ROWS_LAST), :])


_BM = 1000


def _tc_first(x, W1, deg):
    def body(x_ref, w_ref, d_ref, o_ref):
        dinv = lax.rsqrt(d_ref[0] + d_ref[1] + 1.0)
        o_ref[...] = jnp.dot(x_ref[...], w_ref[...],
                             preferred_element_type=jnp.float32) * dinv
    return pl.pallas_call(
        body,
        grid=(N // _BM,),
        in_specs=[
            pl.BlockSpec((_BM, D), lambda i: (i, 0)),
            pl.BlockSpec((D, D), lambda i: (0, 0)),
            pl.BlockSpec((CORES, _BM, 1), lambda i: (0, i, 0)),
        ],
        out_specs=pl.BlockSpec((_BM, D), lambda i: (i, 0)),
        out_shape=jax.ShapeDtypeStruct((N, D), jnp.float32),
    )(x, W1, deg)


def _tc_mid(acc, hp, deg, b, W2):
    def body(a_ref, h_ref, d_ref, b_ref, w_ref, o_ref):
        dinv = lax.rsqrt(d_ref[0] + d_ref[1] + 1.0)
        tot = a_ref[0] + a_ref[1] + h_ref[...]
        x2 = jnp.maximum(tot * dinv + b_ref[...], 0.0)
        o_ref[...] = jnp.dot(x2, w_ref[...],
                             preferred_element_type=jnp.float32) * dinv
    return pl.pallas_call(
        body,
        grid=(N // _BM,),
        in_specs=[
            pl.BlockSpec((CORES, _BM, D), lambda i: (0, i, 0)),
            pl.BlockSpec((_BM, D), lambda i: (i, 0)),
            pl.BlockSpec((CORES, _BM, 1), lambda i: (0, i, 0)),
            pl.BlockSpec((1, D), lambda i: (0, 0)),
            pl.BlockSpec((D, D), lambda i: (0, 0)),
        ],
        out_specs=pl.BlockSpec((_BM, D), lambda i: (i, 0)),
        out_shape=jax.ShapeDtypeStruct((N, D), jnp.float32),
    )(acc, hp, deg, b, W2)


def _tc_last(acc, hp, deg, b, Wfm, Wfx, bfc):
    def body(a_ref, h_ref, d_ref, b_ref, wm_ref, wx_ref, bf_ref, o_ref,
             sum_ref, max_ref):
        i = pl.program_id(0)
        dinv = lax.rsqrt(d_ref[0] + d_ref[1] + 1.0)
        tot = a_ref[0] + a_ref[1] + h_ref[...]
        x3 = jnp.maximum(tot * dinv + b_ref[...], 0.0)
        bsum = jnp.sum(x3, axis=0, keepdims=True)
        bmax = jnp.max(x3, axis=0, keepdims=True)

        @pl.when(i == 0)
        def _init():
            sum_ref[...] = bsum
            max_ref[...] = bmax

        @pl.when(i > 0)
        def _accum():
            sum_ref[...] += bsum
            max_ref[...] = jnp.maximum(max_ref[...], bmax)

        @pl.when(i == N // _BM - 1)
        def _final():
            mean = sum_ref[...] * (1.0 / N)
            o_ref[...] = (jnp.dot(mean, wm_ref[...],
                                  preferred_element_type=jnp.float32)
                          + jnp.dot(max_ref[...], wx_ref[...],
                                    preferred_element_type=jnp.float32)
                          + bf_ref[...])
    return pl.pallas_call(
        body,
        grid=(N // _BM,),
        in_specs=[
            pl.BlockSpec((CORES, _BM, D), lambda i: (0, i, 0)),
            pl.BlockSpec((_BM, D), lambda i: (i, 0)),
            pl.BlockSpec((CORES, _BM, 1), lambda i: (0, i, 0)),
            pl.BlockSpec((1, D), lambda i: (0, 0)),
            pl.BlockSpec((D, D), lambda i: (0, 0)),
            pl.BlockSpec((D, D), lambda i: (0, 0)),
            pl.BlockSpec((1, D), lambda i: (0, 0)),
        ],
        out_specs=pl.BlockSpec((1, D), lambda i: (0, 0)),
        out_shape=jax.ShapeDtypeStruct((1, D), jnp.float32),
        scratch_shapes=[
            pltpu.VMEM((1, D), jnp.float32),
            pltpu.VMEM((1, D), jnp.float32),
        ],
    )(acc, hp, deg, b, Wfm, Wfx, bfc)


def kernel(node_features, edge_index, W1, b1, W2, b2, W_fc, b_fc):
    ei = edge_index.astype(jnp.int32)
    npad = E_PAD - E
    src = jnp.concatenate([ei[0], jnp.zeros((npad,), jnp.int32)])
    dst = jnp.concatenate([ei[1], jnp.full((npad,), PAD_DST, jnp.int32)])
    dst2d = dst.reshape(E_PAD // CHUNK, CHUNK)

    deg = _sc_degree(dst2d)[:, :N].reshape(CORES, N, 1)
    hp1 = _tc_first(node_features, W1, deg)
    acc1 = _sc_message(hp1, src, dst2d)
    hp2 = _tc_mid(acc1, hp1, deg, b1.reshape(1, D), W2)
    acc2 = _sc_message(hp2, src, dst2d)
    return _tc_last(acc2, hp2, deg, b2.reshape(1, D),
                    W_fc[:D], W_fc[D:], b_fc.reshape(1, D))
